# baseline (speedup 1.0000x reference)
import functools

import jax
import jax.numpy as jnp
from jax import lax
from jax.experimental import pallas as pl
from jax.experimental.pallas import tpu as pltpu
from jax.experimental.pallas import tpu_sc as plsc

NC, NS = 2, 16
NW = NC * NS
B, S, D = 4096, 200, 64
ROW = S * D
RPW = B // NW
CH = 1
NCH = RPW // CH
NBUF = 2


def _sc_body(x_hbm, emb_hbm, out_hbm, emb_v, tbuf, spi, spo, *sems):
    isem = sems[:NBUF]
    osem = sems[NBUF:2 * NBUF]
    cid = lax.axis_index("c")
    sid = lax.axis_index("s")
    wid = sid * NC + cid
    base = wid * RPW

    def start_in(b, j):
        pltpu.async_copy(x_hbm.at[pl.ds(base + j * CH, CH)],
                         spi.at[sid, b], isem[b])

    def wait_in(b, j):
        pltpu.make_async_copy(x_hbm.at[pl.ds(base + j * CH, CH)],
                              spi.at[sid, b], isem[b]).wait()

    def start_out(b, j):
        pltpu.async_copy(spo.at[sid, b],
                         out_hbm.at[pl.ds(base + j * CH, CH)], osem[b])

    def wait_out(b, j):
        pltpu.make_async_copy(spo.at[sid, b],
                              out_hbm.at[pl.ds(base + j * CH, CH)],
                              osem[b]).wait()

    def compute():
        for r in range(CH):
            @plsc.parallel_loop(0, ROW, step=16, unroll=8)
            def _(i):
                sl = pl.ds(i, 16)
                xv = tbuf[r, sl]
                ev = emb_v[0, sl]
                tbuf[r, sl] = jnp.where(xv == 0.0, 0.0, xv + ev)

    def step(b, j, first, last):
        wait_in(b, j)
        pltpu.sync_copy(spi.at[sid, b], tbuf)
        if not last:
            start_in(b, j + NBUF)
        compute()
        if not first:
            wait_out(b, j - NBUF)
        pltpu.sync_copy(tbuf, spo.at[sid, b])
        start_out(b, j)

    for b in range(NBUF):
        start_in(b, b)
    pltpu.sync_copy(emb_hbm, emb_v)

    for b in range(NBUF):
        step(b, b, first=True, last=False)

    @pl.loop(NBUF, NCH - NBUF, step=NBUF)
    def _(j0):
        for b in range(NBUF):
            step(b, j0 + b, first=False, last=False)

    for b in range(NBUF):
        step(b, NCH - NBUF + b, first=False, last=True)
    for b in range(NBUF):
        wait_out(b, NCH - NBUF + b)


_scratch = (
    [pltpu.VMEM((1, ROW), jnp.float32),
     pltpu.VMEM((CH, ROW), jnp.float32),
     pltpu.VMEM_SHARED((NS, NBUF, CH, ROW), jnp.float32),
     pltpu.VMEM_SHARED((NS, NBUF, CH, ROW), jnp.float32)]
    + [pltpu.SemaphoreType.DMA for _ in range(2 * NBUF)]
)

_sc_kernel = functools.partial(
    pl.kernel,
    out_type=jax.ShapeDtypeStruct((B, ROW), jnp.float32),
    mesh=plsc.VectorSubcoreMesh(core_axis_name="c", subcore_axis_name="s"),
    scratch_types=_scratch,
)(_sc_body)


def kernel(x, pos_emb):
    out = _sc_kernel(x.reshape(B, ROW), pos_emb.reshape(1, ROW))
    return out.reshape(B, S, D)
